# Initial kernel scaffold; baseline (speedup 1.0000x reference)
import jax
import jax.numpy as jnp
from jax.experimental import pallas as pl


def kernel(source, emb, W_ih, W_hh, b_ih, b_hh):
    raise NotImplementedError("write your pallas kernel here")


# speedup vs baseline: 3.6643x; 3.6643x over previous
import jax
import jax.numpy as jnp
from jax.experimental import pallas as pl
from jax.experimental.pallas import tpu as pltpu

VOCAB_ = 32000
EMB_ = 1024
HID_ = 1024
BATCH_ = 64
SEQ_ = 512

T_BLK = 8
TOK_BLK = T_BLK * BATCH_
N_BLK = SEQ_ // T_BLK
HALF_BLK = N_BLK // 2
HALF_B = BATCH_ // 2


def _proj_kernel(src_ref, emb_ref, w_ref, b_ref, out_ref, gbuf, sems):
    t = pl.program_id(0)
    i = pl.program_id(1)
    nhalf = pl.num_programs(1)
    blk = t * nhalf + i

    def issue(blk_idx, slot):
        base = blk_idx * TOK_BLK
        for mi in range(TOK_BLK):
            tok = src_ref[base + mi]
            pltpu.make_async_copy(
                emb_ref.at[tok], gbuf.at[slot, mi], sems.at[slot]
            ).start()

    @pl.when(i == 0)
    def _():
        issue(blk, 0)

    slot = jax.lax.rem(i, 2)

    @pl.when(i + 1 < nhalf)
    def _():
        issue(blk + 1, 1 - slot)

    pltpu.make_async_copy(gbuf.at[slot], gbuf.at[slot], sems.at[slot]).wait()

    x = gbuf[slot]
    out_ref[...] = (
        jnp.dot(x, w_ref[...], preferred_element_type=jnp.float32) + b_ref[...]
    )


def _cell_kernel(x_ref, w_ref, out_ref, h_ref, c_ref):
    i = pl.program_id(1)

    @pl.when(i == 0)
    def _():
        h_ref[...] = jnp.zeros_like(h_ref)
        c_ref[...] = jnp.zeros_like(c_ref)

    def sig(v):
        return 0.5 * jnp.tanh(0.5 * v) + 0.5

    h = h_ref[...]
    c = c_ref[...]
    for k in range(T_BLK):
        gates = x_ref[k] + jnp.dot(
            h, w_ref[...], preferred_element_type=jnp.float32
        )
        i_g = sig(gates[:, :HID_])
        f_g = sig(gates[:, HID_:2 * HID_])
        g_g = jnp.tanh(gates[:, 2 * HID_:3 * HID_])
        o_g = sig(gates[:, 3 * HID_:])
        c = f_g * c + i_g * g_g
        h = o_g * jnp.tanh(c)
    h_ref[...] = h
    c_ref[...] = c

    @pl.when(i == pl.num_programs(1) - 1)
    def _():
        out_ref[0] = h
        out_ref[1] = c


def _proj_call(src_flat, emb, w_ihT, bias, *, interpret=False):
    return pl.pallas_call(
        _proj_kernel,
        out_shape=jax.ShapeDtypeStruct((SEQ_ * BATCH_, 4 * HID_), jnp.float32),
        grid_spec=pltpu.PrefetchScalarGridSpec(
            num_scalar_prefetch=1,
            grid=(2, HALF_BLK),
            in_specs=[
                pl.BlockSpec(memory_space=pl.ANY),
                pl.BlockSpec((EMB_, 4 * HID_), lambda t, i, s: (0, 0)),
                pl.BlockSpec((1, 4 * HID_), lambda t, i, s: (0, 0)),
            ],
            out_specs=pl.BlockSpec(
                (TOK_BLK, 4 * HID_), lambda t, i, s: (t * HALF_BLK + i, 0)
            ),
            scratch_shapes=[
                pltpu.VMEM((2, TOK_BLK, EMB_), jnp.float32),
                pltpu.SemaphoreType.DMA((2,)),
            ],
        ),
        compiler_params=pltpu.CompilerParams(
            dimension_semantics=("parallel", "arbitrary"),
            vmem_limit_bytes=56 * 1024 * 1024,
        ),
        name="embed_proj",
        interpret=interpret,
    )(src_flat, emb, w_ihT, bias)


def _cell_call(x3, w_hhT, *, interpret=False):
    return pl.pallas_call(
        _cell_kernel,
        out_shape=jax.ShapeDtypeStruct((2, BATCH_, HID_), jnp.float32),
        grid=(2, N_BLK),
        in_specs=[
            pl.BlockSpec((T_BLK, HALF_B, 4 * HID_), lambda t, i: (i, t, 0)),
            pl.BlockSpec((HID_, 4 * HID_), lambda t, i: (0, 0)),
        ],
        out_specs=pl.BlockSpec((2, HALF_B, HID_), lambda t, i: (0, t, 0)),
        scratch_shapes=[
            pltpu.VMEM((HALF_B, HID_), jnp.float32),
            pltpu.VMEM((HALF_B, HID_), jnp.float32),
        ],
        compiler_params=pltpu.CompilerParams(
            dimension_semantics=("parallel", "arbitrary"),
            vmem_limit_bytes=40 * 1024 * 1024,
        ),
        name="lstm_cell",
        interpret=interpret,
    )(x3, w_hhT)


def kernel(source, emb, W_ih, W_hh, b_ih, b_hh, *, interpret=False):
    src_flat = jnp.transpose(source).reshape(-1)
    w_ihT = jnp.transpose(W_ih)
    w_hhT = jnp.transpose(W_hh)
    bias = (b_ih + b_hh).reshape(1, 4 * HID_)
    x = _proj_call(src_flat, emb, w_ihT, bias, interpret=interpret)
    x3 = x.reshape(SEQ_, BATCH_, 4 * HID_)
    return _cell_call(x3, w_hhT, interpret=interpret)


# speedup vs baseline: 5.3230x; 1.4527x over previous
import jax
import jax.numpy as jnp
from jax.experimental import pallas as pl
from jax.experimental.pallas import tpu as pltpu

VOCAB_ = 32000
EMB_ = 1024
HID_ = 1024
BATCH_ = 64
SEQ_ = 512

T_BLK = 8
TOK_BLK = T_BLK * BATCH_
N_BLK = SEQ_ // T_BLK


def _proj_kernel(src_ref, emb_ref, w_ref, b_ref, out_ref, gbuf, sems):
    j = pl.program_id(0)
    nblk = pl.num_programs(0)

    def issue(blk_idx, slot):
        base = blk_idx * TOK_BLK
        for mi in range(TOK_BLK):
            tok = src_ref[base + mi]
            pltpu.make_async_copy(
                emb_ref.at[tok], gbuf.at[slot, mi], sems.at[slot]
            ).start()

    @pl.when(j == 0)
    def _():
        issue(0, 0)

    slot = jax.lax.rem(j, 2)

    pltpu.make_async_copy(gbuf.at[slot], gbuf.at[slot], sems.at[slot]).wait()

    x = gbuf[slot]
    out_ref[...] = (
        jax.lax.dot_general(
            x, w_ref[...],
            dimension_numbers=(((1,), (1,)), ((), ())),
            preferred_element_type=jnp.float32,
        )
        + b_ref[...]
    )

    @pl.when(j + 1 < nblk)
    def _():
        issue(j + 1, 1 - slot)


def _cell_kernel(x_ref, w_ref, out_ref, h_ref, c_ref):
    i = pl.program_id(0)

    @pl.when(i == 0)
    def _():
        h_ref[...] = jnp.zeros_like(h_ref)
        c_ref[...] = jnp.zeros_like(c_ref)

    def sig(v):
        return 0.5 * jnp.tanh(0.5 * v) + 0.5

    h = h_ref[...]
    c = c_ref[...]
    for k in range(T_BLK):
        gates = x_ref[k] + jnp.dot(
            h, w_ref[...], preferred_element_type=jnp.float32
        )
        i_g = sig(gates[:, :HID_])
        f_g = sig(gates[:, HID_:2 * HID_])
        g_g = jnp.tanh(gates[:, 2 * HID_:3 * HID_])
        o_g = sig(gates[:, 3 * HID_:])
        c = f_g * c + i_g * g_g
        h = o_g * jnp.tanh(c)
    h_ref[...] = h
    c_ref[...] = c

    @pl.when(i == pl.num_programs(0) - 1)
    def _():
        out_ref[0] = h
        out_ref[1] = c


def _proj_call(src_flat, emb, w_ih, bias, *, interpret=False):
    return pl.pallas_call(
        _proj_kernel,
        out_shape=jax.ShapeDtypeStruct((SEQ_ * BATCH_, 4 * HID_), jnp.float32),
        grid_spec=pltpu.PrefetchScalarGridSpec(
            num_scalar_prefetch=1,
            grid=(N_BLK,),
            in_specs=[
                pl.BlockSpec(memory_space=pl.ANY),
                pl.BlockSpec((4 * HID_, EMB_), lambda j, s: (0, 0)),
                pl.BlockSpec((1, 4 * HID_), lambda j, s: (0, 0)),
            ],
            out_specs=pl.BlockSpec((TOK_BLK, 4 * HID_), lambda j, s: (j, 0)),
            scratch_shapes=[
                pltpu.VMEM((2, TOK_BLK, EMB_), jnp.float32),
                pltpu.SemaphoreType.DMA((2,)),
            ],
        ),
        compiler_params=pltpu.CompilerParams(
            dimension_semantics=("arbitrary",),
            vmem_limit_bytes=56 * 1024 * 1024,
        ),
        name="embed_proj",
        interpret=interpret,
    )(src_flat, emb, w_ih, bias)


def _cell_call(x3, w_hhT, *, interpret=False):
    return pl.pallas_call(
        _cell_kernel,
        out_shape=jax.ShapeDtypeStruct((2, BATCH_, HID_), jnp.float32),
        grid=(N_BLK,),
        in_specs=[
            pl.BlockSpec((T_BLK, BATCH_, 4 * HID_), lambda i: (i, 0, 0)),
            pl.BlockSpec((HID_, 4 * HID_), lambda i: (0, 0)),
        ],
        out_specs=pl.BlockSpec((2, BATCH_, HID_), lambda i: (0, 0, 0)),
        scratch_shapes=[
            pltpu.VMEM((BATCH_, HID_), jnp.float32),
            pltpu.VMEM((BATCH_, HID_), jnp.float32),
        ],
        compiler_params=pltpu.CompilerParams(
            dimension_semantics=("arbitrary",),
            vmem_limit_bytes=48 * 1024 * 1024,
        ),
        name="lstm_cell",
        interpret=interpret,
    )(x3, w_hhT)


def kernel(source, emb, W_ih, W_hh, b_ih, b_hh, *, interpret=False):
    src_flat = jnp.transpose(source).reshape(-1)
    w_hhT = jnp.transpose(W_hh)
    bias = (b_ih + b_hh).reshape(1, 4 * HID_)
    x = _proj_call(src_flat, emb, W_ih, bias, interpret=interpret)
    x3 = x.reshape(SEQ_, BATCH_, 4 * HID_)
    return _cell_call(x3, w_hhT, interpret=interpret)


# speedup vs baseline: 6.2811x; 1.1800x over previous
import jax
import jax.numpy as jnp
from jax.experimental import pallas as pl
from jax.experimental.pallas import tpu as pltpu

VOCAB_ = 32000
EMB_ = 1024
HID_ = 1024
BATCH_ = 64
SEQ_ = 512

T_BLK = 8
TOK_BLK = T_BLK * BATCH_
N_BLK = SEQ_ // T_BLK


def _proj_kernel(src_ref, emb_ref, w_ref, b_ref, out_ref, gbuf0, gbuf1, sems):
    j = pl.program_id(0)
    nblk = pl.num_programs(0)
    bufs = (gbuf0, gbuf1)

    def issue(blk_idx, slot):
        buf = bufs[slot]
        base = blk_idx * TOK_BLK
        for mi in range(TOK_BLK):
            tok = src_ref[base + mi]
            pltpu.make_async_copy(
                emb_ref.at[tok], buf.at[mi], sems.at[slot]
            ).start()

    @pl.when(j == 0)
    def _():
        issue(0, 0)

    def step(slot):
        buf = bufs[slot]
        pltpu.make_async_copy(buf, buf, sems.at[slot]).wait()
        out_ref[...] = (
            jax.lax.dot_general(
                buf[...], w_ref[...],
                dimension_numbers=(((1,), (1,)), ((), ())),
                preferred_element_type=jnp.float32,
            )
            + b_ref[...]
        )
        @pl.when(j + 1 < nblk)
        def _():
            issue(j + 1, 1 - slot)

    @pl.when(jax.lax.rem(j, 2) == 0)
    def _():
        step(0)

    @pl.when(jax.lax.rem(j, 2) == 1)
    def _():
        step(1)


def _cell_kernel(x_ref, w_ref, out_ref, h_ref, c_ref):
    i = pl.program_id(0)

    @pl.when(i == 0)
    def _():
        h_ref[...] = jnp.zeros_like(h_ref)
        c_ref[...] = jnp.zeros_like(c_ref)

    def sig(v):
        return 0.5 * jnp.tanh(0.5 * v) + 0.5

    h = h_ref[...]
    c = c_ref[...]
    for k in range(T_BLK):
        gates = x_ref[k] + jnp.dot(
            h, w_ref[...], preferred_element_type=jnp.float32
        )
        i_g = sig(gates[:, :HID_])
        f_g = sig(gates[:, HID_:2 * HID_])
        g_g = jnp.tanh(gates[:, 2 * HID_:3 * HID_])
        o_g = sig(gates[:, 3 * HID_:])
        c = f_g * c + i_g * g_g
        h = o_g * jnp.tanh(c)
    h_ref[...] = h
    c_ref[...] = c

    @pl.when(i == pl.num_programs(0) - 1)
    def _():
        out_ref[0] = h
        out_ref[1] = c


def _proj_call(src_flat, emb, w_ih, bias, *, interpret=False):
    return pl.pallas_call(
        _proj_kernel,
        out_shape=jax.ShapeDtypeStruct((SEQ_ * BATCH_, 4 * HID_), jnp.float32),
        grid_spec=pltpu.PrefetchScalarGridSpec(
            num_scalar_prefetch=1,
            grid=(N_BLK,),
            in_specs=[
                pl.BlockSpec(memory_space=pl.ANY),
                pl.BlockSpec((4 * HID_, EMB_), lambda j, s: (0, 0)),
                pl.BlockSpec((1, 4 * HID_), lambda j, s: (0, 0)),
            ],
            out_specs=pl.BlockSpec((TOK_BLK, 4 * HID_), lambda j, s: (j, 0)),
            scratch_shapes=[
                pltpu.VMEM((TOK_BLK, EMB_), jnp.float32),
                pltpu.VMEM((TOK_BLK, EMB_), jnp.float32),
                pltpu.SemaphoreType.DMA((2,)),
            ],
        ),
        compiler_params=pltpu.CompilerParams(
            dimension_semantics=("arbitrary",),
            vmem_limit_bytes=56 * 1024 * 1024,
        ),
        name="embed_proj",
        interpret=interpret,
    )(src_flat, emb, w_ih, bias)


def _cell_call(x3, w_hhT, *, interpret=False):
    return pl.pallas_call(
        _cell_kernel,
        out_shape=jax.ShapeDtypeStruct((2, BATCH_, HID_), jnp.float32),
        grid=(N_BLK,),
        in_specs=[
            pl.BlockSpec((T_BLK, BATCH_, 4 * HID_), lambda i: (i, 0, 0)),
            pl.BlockSpec((HID_, 4 * HID_), lambda i: (0, 0)),
        ],
        out_specs=pl.BlockSpec((2, BATCH_, HID_), lambda i: (0, 0, 0)),
        scratch_shapes=[
            pltpu.VMEM((BATCH_, HID_), jnp.float32),
            pltpu.VMEM((BATCH_, HID_), jnp.float32),
        ],
        compiler_params=pltpu.CompilerParams(
            dimension_semantics=("arbitrary",),
            vmem_limit_bytes=48 * 1024 * 1024,
        ),
        name="lstm_cell",
        interpret=interpret,
    )(x3, w_hhT)


def kernel(source, emb, W_ih, W_hh, b_ih, b_hh, *, interpret=False):
    src_flat = jnp.transpose(source).reshape(-1)
    w_hhT = jnp.transpose(W_hh)
    bias = (b_ih + b_hh).reshape(1, 4 * HID_)
    x = _proj_call(src_flat, emb, W_ih, bias, interpret=interpret)
    x3 = x.reshape(SEQ_, BATCH_, 4 * HID_)
    return _cell_call(x3, w_hhT, interpret=interpret)


# speedup vs baseline: 6.3420x; 1.0097x over previous
import jax
import jax.numpy as jnp
from jax.experimental import pallas as pl
from jax.experimental.pallas import tpu as pltpu

VOCAB_ = 32000
EMB_ = 1024
HID_ = 1024
BATCH_ = 64
SEQ_ = 512

T_BLK = 8
TOK_BLK = T_BLK * BATCH_
N_BLK = SEQ_ // T_BLK


def _fused_kernel(src_ref, emb_ref, wi_ref, wh_ref, b_ref, out_ref,
                  gbuf0, gbuf1, gbuf2, xbuf, h_ref, c_ref, sems):
    j = pl.program_id(0)
    nblk = pl.num_programs(0)
    bufs = (gbuf0, gbuf1, gbuf2)

    def issue(base, slot):
        buf = bufs[slot]
        for mi in range(TOK_BLK):
            tok = src_ref[base + mi]
            pltpu.make_async_copy(
                emb_ref.at[tok], buf.at[mi], sems.at[slot]
            ).start()

    @pl.when(j == 0)
    def _():
        h_ref[...] = jnp.zeros_like(h_ref)
        c_ref[...] = jnp.zeros_like(c_ref)
        issue(0, 0)
        issue(TOK_BLK, 1)

    nxt_base = jnp.minimum(j + 2, nblk - 1) * TOK_BLK

    def step(slot):
        buf = bufs[slot]
        pltpu.make_async_copy(buf, buf, sems.at[slot]).wait()
        issue(nxt_base, (slot + 2) % 3)
        xbuf[...] = (
            jax.lax.dot_general(
                buf[...], wi_ref[...],
                dimension_numbers=(((1,), (1,)), ((), ())),
                preferred_element_type=jnp.float32,
            )
            + b_ref[...]
        )

    @pl.when(jax.lax.rem(j, 3) == 0)
    def _():
        step(0)

    @pl.when(jax.lax.rem(j, 3) == 1)
    def _():
        step(1)

    @pl.when(jax.lax.rem(j, 3) == 2)
    def _():
        step(2)

    def sig(v):
        return 0.5 * jnp.tanh(0.5 * v) + 0.5

    h = h_ref[...]
    c = c_ref[...]
    for k in range(T_BLK):
        gates = xbuf[pl.ds(k * BATCH_, BATCH_)] + jnp.dot(
            h, wh_ref[...], preferred_element_type=jnp.float32
        )
        i_g = sig(gates[:, :HID_])
        f_g = sig(gates[:, HID_:2 * HID_])
        g_g = jnp.tanh(gates[:, 2 * HID_:3 * HID_])
        o_g = sig(gates[:, 3 * HID_:])
        c = f_g * c + i_g * g_g
        h = o_g * jnp.tanh(c)
    h_ref[...] = h
    c_ref[...] = c

    @pl.when(j == nblk - 1)
    def _():
        out_ref[0] = h
        out_ref[1] = c
        a = N_BLK % 3
        b = (N_BLK + 1) % 3
        pltpu.make_async_copy(bufs[a], bufs[a], sems.at[a]).wait()
        pltpu.make_async_copy(bufs[b], bufs[b], sems.at[b]).wait()


def _fused_call(src_flat, emb, w_ih, w_hhT, bias, *, interpret=False):
    return pl.pallas_call(
        _fused_kernel,
        out_shape=jax.ShapeDtypeStruct((2, BATCH_, HID_), jnp.float32),
        grid_spec=pltpu.PrefetchScalarGridSpec(
            num_scalar_prefetch=1,
            grid=(N_BLK,),
            in_specs=[
                pl.BlockSpec(memory_space=pl.ANY),
                pl.BlockSpec((4 * HID_, EMB_), lambda j, s: (0, 0)),
                pl.BlockSpec((HID_, 4 * HID_), lambda j, s: (0, 0)),
                pl.BlockSpec((1, 4 * HID_), lambda j, s: (0, 0)),
            ],
            out_specs=pl.BlockSpec((2, BATCH_, HID_), lambda j, s: (0, 0, 0)),
            scratch_shapes=[
                pltpu.VMEM((TOK_BLK, EMB_), jnp.float32),
                pltpu.VMEM((TOK_BLK, EMB_), jnp.float32),
                pltpu.VMEM((TOK_BLK, EMB_), jnp.float32),
                pltpu.VMEM((TOK_BLK, 4 * HID_), jnp.float32),
                pltpu.VMEM((BATCH_, HID_), jnp.float32),
                pltpu.VMEM((BATCH_, HID_), jnp.float32),
                pltpu.SemaphoreType.DMA((3,)),
            ],
        ),
        compiler_params=pltpu.CompilerParams(
            dimension_semantics=("arbitrary",),
            vmem_limit_bytes=58 * 1024 * 1024,
        ),
        name="lstm_fused",
        interpret=interpret,
    )(src_flat, emb, w_ih, w_hhT, bias)


def kernel(source, emb, W_ih, W_hh, b_ih, b_hh, *, interpret=False):
    src_flat = jnp.transpose(source).reshape(-1)
    w_hhT = jnp.transpose(W_hh)
    bias = (b_ih + b_hh).reshape(1, 4 * HID_)
    return _fused_call(src_flat, emb, W_ih, w_hhT, bias, interpret=interpret)
